# Initial kernel scaffold; baseline (speedup 1.0000x reference)
import jax
import jax.numpy as jnp
from jax.experimental import pallas as pl


def kernel(src_node_features, src_node_pos, dst_node_features, dst_node_pos, W_src, b_src, W_dst, b_dst, W_ed, b_ed, W_e1, b_e1, W_e2, b_e2, ln_gamma, ln_beta, W_out, b_out):
    raise NotImplementedError("write your pallas kernel here")


# speedup vs baseline: 6.9238x; 6.9238x over previous
import jax
import jax.numpy as jnp
from jax.experimental import pallas as pl
from jax.experimental.pallas import tpu as pltpu

_THR = 5.0
_SRC_SUB = 16
_DST_BLK = 256
_POS_PAD = 128


def _prep_kernel(sf_ref, sp_ref, df_ref, dp_ref,
                 W_src_ref, b_src_ref, W_dst_ref, b_dst_ref,
                 W_ed_ref, W_e1a_ref, b_e1_ref, W_e1c_ref,
                 A_ref, u_ref, C_ref, v_ref, dst_enc_ref):
    src_enc = jnp.maximum(sf_ref[...] @ W_src_ref[...] + b_src_ref[...], 0.0)
    A_ref[...] = src_enc @ W_e1a_ref[...] + b_e1_ref[...]
    sp = sp_ref[...]
    W_ed = W_ed_ref[...]
    u_ref[...] = sp[:, 0:1] * W_ed[0:1, :] + sp[:, 1:2] * W_ed[1:2, :]
    dst_enc = jnp.maximum(df_ref[...] @ W_dst_ref[...] + b_dst_ref[...], 0.0)
    dst_enc_ref[...] = dst_enc
    C_ref[...] = dst_enc @ W_e1c_ref[...]
    dp = dp_ref[...]
    v_ref[...] = dp[:, 0:1] * W_ed[0:1, :] + dp[:, 1:2] * W_ed[1:2, :]


def _main_kernel(A_ref, u_ref, spad_ref, C_ref, v_ref, dpT_ref,
                 dst_enc_ref, dst_feat_ref,
                 W_e1b_ref, b_ed_ref, W_e2_ref, b_e2_ref,
                 ln_g_ref, ln_b_ref, W_out_ref, b_out_ref,
                 out_ref, S_acc, cnt_acc):
    s = pl.program_id(1)
    ns = pl.num_programs(1)
    d = A_ref.shape[1]

    @pl.when(s == 0)
    def _zero():
        S_acc[...] = jnp.zeros_like(S_acc)
        cnt_acc[...] = jnp.zeros_like(cnt_acc)

    sx = spad_ref[:, 0:1]
    sy = spad_ref[:, 1:2]
    dx = dpT_ref[0:1, :]
    dy = dpT_ref[1:2, :]
    diffx = sx - dx
    diffy = sy - dy
    dist = jnp.sqrt(diffx * diffx + diffy * diffy)
    mask = (dist <= _THR).astype(jnp.float32)

    u = u_ref[...]
    v = v_ref[...]
    b_ed = b_ed_ref[...].reshape(1, 1, d)
    ed = jnp.maximum(u[:, None, :] - v[None, :, :] + b_ed, 0.0)
    M = jax.lax.dot_general(
        ed.reshape(_SRC_SUB * _DST_BLK, d), W_e1b_ref[...],
        (((1,), (0,)), ((), ())), preferred_element_type=jnp.float32)
    hidden = jnp.maximum(
        M.reshape(_SRC_SUB, _DST_BLK, d)
        + A_ref[...][:, None, :] + C_ref[...][None, :, :], 0.0)
    S_acc[...] += jnp.sum(mask[:, :, None] * hidden, axis=0)
    ones = jnp.ones((_SRC_SUB, 128), jnp.float32)
    cnt_acc[...] += jax.lax.dot_general(
        mask, ones, (((0,), (0,)), ((), ())),
        preferred_element_type=jnp.float32)

    @pl.when(s == ns - 1)
    def _finalize():
        cnt_col = cnt_acc[:, 0:1]
        acc = (dst_enc_ref[...] + S_acc[...] @ W_e2_ref[...]
               + cnt_col * b_e2_ref[...])
        mean = jnp.mean(acc, axis=1, keepdims=True)
        cen = acc - mean
        var = jnp.mean(cen * cen, axis=1, keepdims=True)
        nrm = cen / jnp.sqrt(var + 1e-5) * ln_g_ref[...] + ln_b_ref[...]
        h = jnp.maximum(nrm, 0.0)
        o = h @ W_out_ref[...] + b_out_ref[...] + dst_feat_ref[...]
        out_ref[...] = jnp.maximum(o, 0.0)


@jax.jit
def kernel(src_node_features, src_node_pos, dst_node_features, dst_node_pos,
           W_src, b_src, W_dst, b_dst, W_ed, b_ed,
           W_e1, b_e1, W_e2, b_e2, ln_gamma, ln_beta, W_out, b_out):
    src_n, d = src_node_features.shape
    dst_n = dst_node_features.shape[0]
    f32 = jnp.float32

    W_e1a = W_e1[0:d]
    W_e1b = W_e1[d:2 * d]
    W_e1c = W_e1[2 * d:]
    b_src_r = b_src.reshape(1, d)
    b_dst_r = b_dst.reshape(1, d)
    b_e1_r = b_e1.reshape(1, d)
    b_ed_r = b_ed.reshape(1, d)
    b_e2_r = b_e2.reshape(1, d)
    ln_g_r = ln_gamma.reshape(1, d)
    ln_b_r = ln_beta.reshape(1, d)
    b_out_r = b_out.reshape(1, d)

    spad = jnp.pad(src_node_pos, ((0, 0), (0, _POS_PAD - 2)))
    dpT = jnp.pad(dst_node_pos.T, ((0, 6), (0, 0)))

    A, u, C, v, dst_enc = pl.pallas_call(
        _prep_kernel,
        out_shape=[
            jax.ShapeDtypeStruct((src_n, d), f32),
            jax.ShapeDtypeStruct((src_n, d), f32),
            jax.ShapeDtypeStruct((dst_n, d), f32),
            jax.ShapeDtypeStruct((dst_n, d), f32),
            jax.ShapeDtypeStruct((dst_n, d), f32),
        ],
    )(src_node_features, src_node_pos, dst_node_features, dst_node_pos,
      W_src, b_src_r, W_dst, b_dst_r, W_ed, W_e1a, b_e1_r, W_e1c)

    db = dst_n // _DST_BLK
    sb = src_n // _SRC_SUB
    out = pl.pallas_call(
        _main_kernel,
        grid=(db, sb),
        in_specs=[
            pl.BlockSpec((_SRC_SUB, d), lambda i, j: (j, 0)),
            pl.BlockSpec((_SRC_SUB, d), lambda i, j: (j, 0)),
            pl.BlockSpec((_SRC_SUB, _POS_PAD), lambda i, j: (j, 0)),
            pl.BlockSpec((_DST_BLK, d), lambda i, j: (i, 0)),
            pl.BlockSpec((_DST_BLK, d), lambda i, j: (i, 0)),
            pl.BlockSpec((8, _DST_BLK), lambda i, j: (0, i)),
            pl.BlockSpec((_DST_BLK, d), lambda i, j: (i, 0)),
            pl.BlockSpec((_DST_BLK, d), lambda i, j: (i, 0)),
            pl.BlockSpec((d, d), lambda i, j: (0, 0)),
            pl.BlockSpec((1, d), lambda i, j: (0, 0)),
            pl.BlockSpec((d, d), lambda i, j: (0, 0)),
            pl.BlockSpec((1, d), lambda i, j: (0, 0)),
            pl.BlockSpec((1, d), lambda i, j: (0, 0)),
            pl.BlockSpec((1, d), lambda i, j: (0, 0)),
            pl.BlockSpec((d, d), lambda i, j: (0, 0)),
            pl.BlockSpec((1, d), lambda i, j: (0, 0)),
        ],
        out_specs=pl.BlockSpec((_DST_BLK, d), lambda i, j: (i, 0)),
        out_shape=jax.ShapeDtypeStruct((dst_n, d), f32),
        scratch_shapes=[
            pltpu.VMEM((_DST_BLK, d), f32),
            pltpu.VMEM((_DST_BLK, 128), f32),
        ],
        compiler_params=pltpu.CompilerParams(
            dimension_semantics=("parallel", "arbitrary")),
    )(A, u, spad, C, v, dpT, dst_enc, dst_node_features,
      W_e1b, b_ed_r, W_e2, b_e2_r, ln_g_r, ln_b_r, W_out, b_out_r)
    return out


# speedup vs baseline: 14.9706x; 2.1622x over previous
import jax
import jax.numpy as jnp
from jax.experimental import pallas as pl
from jax.experimental.pallas import tpu as pltpu

_THR = 5.0
_SRC_SUB = 16
_DST_BLK = 256
_POS_PAD = 128


def _prep_kernel(sf_ref, sp_ref, df_ref, dp_ref,
                 W_src_ref, b_src_ref, W_dst_ref, b_dst_ref,
                 W_ed_ref, b_ed_ref, W_e1a_ref, b_e1_ref, W_e1c_ref,
                 A_ref, u_ref, C_ref, v_ref, dst_enc_ref):
    src_enc = jnp.maximum(sf_ref[...] @ W_src_ref[...] + b_src_ref[...], 0.0)
    A_ref[...] = src_enc @ W_e1a_ref[...] + b_e1_ref[...]
    sp = sp_ref[...]
    W_ed = W_ed_ref[...]
    u_ref[...] = (sp[:, 0:1] * W_ed[0:1, :] + sp[:, 1:2] * W_ed[1:2, :]
                  + b_ed_ref[...])
    dst_enc = jnp.maximum(df_ref[...] @ W_dst_ref[...] + b_dst_ref[...], 0.0)
    dst_enc_ref[...] = dst_enc
    C_ref[...] = dst_enc @ W_e1c_ref[...]
    dp = dp_ref[...]
    v_ref[...] = dp[:, 0:1] * W_ed[0:1, :] + dp[:, 1:2] * W_ed[1:2, :]


def _main_kernel(A_ref, u_ref, spad_ref, C_ref, v_ref, dpT_ref,
                 dst_enc_ref, dst_feat_ref,
                 W_e1b_ref, W_e2_ref, b_e2_ref,
                 ln_g_ref, ln_b_ref, W_out_ref, b_out_ref,
                 out_ref, S_acc, cnt_acc):
    s = pl.program_id(1)
    ns = pl.num_programs(1)
    d = A_ref.shape[1]

    @pl.when(s == 0)
    def _zero():
        S_acc[...] = jnp.zeros_like(S_acc)
        cnt_acc[...] = jnp.zeros_like(cnt_acc)

    sx = spad_ref[:, 0:1]
    sy = spad_ref[:, 1:2]
    dx = dpT_ref[0:1, :]
    dy = dpT_ref[1:2, :]

    overlap = jnp.logical_and(jnp.min(sx) <= jnp.max(dx) + _THR,
                              jnp.max(sx) >= jnp.min(dx) - _THR)

    @pl.when(overlap)
    def _tile():
        diffx = sx - dx
        diffy = sy - dy
        dist = jnp.sqrt(diffx * diffx + diffy * diffy)
        mask = (dist <= _THR).astype(jnp.float32)

        u = u_ref[...]
        v = v_ref[...]
        ed = jnp.maximum(u[:, None, :] - v[None, :, :], 0.0)
        M = jax.lax.dot_general(
            ed.reshape(_SRC_SUB * _DST_BLK, d), W_e1b_ref[...],
            (((1,), (0,)), ((), ())), preferred_element_type=jnp.float32)
        hidden = jnp.maximum(
            M.reshape(_SRC_SUB, _DST_BLK, d)
            + A_ref[...][:, None, :] + C_ref[...][None, :, :], 0.0)
        S_acc[...] += jnp.sum(mask[:, :, None] * hidden, axis=0)
        ones = jnp.ones((_SRC_SUB, 128), jnp.float32)
        cnt_acc[...] += jax.lax.dot_general(
            mask, ones, (((0,), (0,)), ((), ())),
            preferred_element_type=jnp.float32)

    @pl.when(s == ns - 1)
    def _finalize():
        cnt_col = cnt_acc[:, 0:1]
        acc = (dst_enc_ref[...] + S_acc[...] @ W_e2_ref[...]
               + cnt_col * b_e2_ref[...])
        mean = jnp.mean(acc, axis=1, keepdims=True)
        cen = acc - mean
        var = jnp.mean(cen * cen, axis=1, keepdims=True)
        nrm = cen / jnp.sqrt(var + 1e-5) * ln_g_ref[...] + ln_b_ref[...]
        h = jnp.maximum(nrm, 0.0)
        o = h @ W_out_ref[...] + b_out_ref[...] + dst_feat_ref[...]
        out_ref[...] = jnp.maximum(o, 0.0)


@jax.jit
def kernel(src_node_features, src_node_pos, dst_node_features, dst_node_pos,
           W_src, b_src, W_dst, b_dst, W_ed, b_ed,
           W_e1, b_e1, W_e2, b_e2, ln_gamma, ln_beta, W_out, b_out):
    src_n, d = src_node_features.shape
    dst_n = dst_node_features.shape[0]
    f32 = jnp.float32

    W_e1a = W_e1[0:d]
    W_e1b = W_e1[d:2 * d]
    W_e1c = W_e1[2 * d:]
    b_src_r = b_src.reshape(1, d)
    b_dst_r = b_dst.reshape(1, d)
    b_e1_r = b_e1.reshape(1, d)
    b_ed_r = b_ed.reshape(1, d)
    b_e2_r = b_e2.reshape(1, d)
    ln_g_r = ln_gamma.reshape(1, d)
    ln_b_r = ln_beta.reshape(1, d)
    b_out_r = b_out.reshape(1, d)

    sperm = jnp.argsort(src_node_pos[:, 0])
    dperm = jnp.argsort(dst_node_pos[:, 0])
    sf_s = jnp.take(src_node_features, sperm, axis=0)
    sp_s = jnp.take(src_node_pos, sperm, axis=0)
    df_s = jnp.take(dst_node_features, dperm, axis=0)
    dp_s = jnp.take(dst_node_pos, dperm, axis=0)

    spad = jnp.pad(sp_s, ((0, 0), (0, _POS_PAD - 2)))
    dpT = jnp.pad(dp_s.T, ((0, 6), (0, 0)))

    A, u, C, v, dst_enc = pl.pallas_call(
        _prep_kernel,
        out_shape=[
            jax.ShapeDtypeStruct((src_n, d), f32),
            jax.ShapeDtypeStruct((src_n, d), f32),
            jax.ShapeDtypeStruct((dst_n, d), f32),
            jax.ShapeDtypeStruct((dst_n, d), f32),
            jax.ShapeDtypeStruct((dst_n, d), f32),
        ],
    )(sf_s, sp_s, df_s, dp_s,
      W_src, b_src_r, W_dst, b_dst_r, W_ed, b_ed_r, W_e1a, b_e1_r, W_e1c)

    db = dst_n // _DST_BLK
    sb = src_n // _SRC_SUB
    out = pl.pallas_call(
        _main_kernel,
        grid=(db, sb),
        in_specs=[
            pl.BlockSpec((_SRC_SUB, d), lambda i, j: (j, 0)),
            pl.BlockSpec((_SRC_SUB, d), lambda i, j: (j, 0)),
            pl.BlockSpec((_SRC_SUB, _POS_PAD), lambda i, j: (j, 0)),
            pl.BlockSpec((_DST_BLK, d), lambda i, j: (i, 0)),
            pl.BlockSpec((_DST_BLK, d), lambda i, j: (i, 0)),
            pl.BlockSpec((8, _DST_BLK), lambda i, j: (0, i)),
            pl.BlockSpec((_DST_BLK, d), lambda i, j: (i, 0)),
            pl.BlockSpec((_DST_BLK, d), lambda i, j: (i, 0)),
            pl.BlockSpec((d, d), lambda i, j: (0, 0)),
            pl.BlockSpec((d, d), lambda i, j: (0, 0)),
            pl.BlockSpec((1, d), lambda i, j: (0, 0)),
            pl.BlockSpec((1, d), lambda i, j: (0, 0)),
            pl.BlockSpec((1, d), lambda i, j: (0, 0)),
            pl.BlockSpec((d, d), lambda i, j: (0, 0)),
            pl.BlockSpec((1, d), lambda i, j: (0, 0)),
        ],
        out_specs=pl.BlockSpec((_DST_BLK, d), lambda i, j: (i, 0)),
        out_shape=jax.ShapeDtypeStruct((dst_n, d), f32),
        scratch_shapes=[
            pltpu.VMEM((_DST_BLK, d), f32),
            pltpu.VMEM((_DST_BLK, 128), f32),
        ],
        compiler_params=pltpu.CompilerParams(
            dimension_semantics=("parallel", "arbitrary")),
    )(A, u, spad, C, v, dpT, dst_enc, df_s,
      W_e1b, W_e2, b_e2_r, ln_g_r, ln_b_r, W_out, b_out_r)
    return jnp.take(out, jnp.argsort(dperm), axis=0)


# speedup vs baseline: 22.4871x; 1.5021x over previous
import jax
import jax.numpy as jnp
from jax.experimental import pallas as pl
from jax.experimental.pallas import tpu as pltpu

_THR = 5.0
_SRC_SUB = 32
_DST_BLK = 256
_POS_PAD = 128


def _prep_kernel(sf_ref, sp_ref, df_ref, dp_ref,
                 W_src_ref, b_src_ref, W_dst_ref, b_dst_ref,
                 W_ed_ref, b_ed_ref, W_e1a_ref, b_e1_ref, W_e1c_ref,
                 A_ref, u_ref, C_ref, v_ref, dst_enc_ref):
    src_enc = jnp.maximum(sf_ref[...] @ W_src_ref[...] + b_src_ref[...], 0.0)
    A_ref[...] = src_enc @ W_e1a_ref[...] + b_e1_ref[...]
    sp = sp_ref[...]
    W_ed = W_ed_ref[...]
    u_ref[...] = (sp[:, 0:1] * W_ed[0:1, :] + sp[:, 1:2] * W_ed[1:2, :]
                  + b_ed_ref[...])
    dst_enc = jnp.maximum(df_ref[...] @ W_dst_ref[...] + b_dst_ref[...], 0.0)
    dst_enc_ref[...] = dst_enc
    C_ref[...] = dst_enc @ W_e1c_ref[...]
    dp = dp_ref[...]
    v_ref[...] = dp[:, 0:1] * W_ed[0:1, :] + dp[:, 1:2] * W_ed[1:2, :]


def _main_kernel(A_ref, u_ref, spad_ref, C_ref, v_ref, dpT_ref,
                 dst_enc_ref, dst_feat_ref,
                 W_e1b_ref, W_e2_ref, b_e2_ref,
                 ln_g_ref, ln_b_ref, W_out_ref, b_out_ref,
                 out_ref, S_acc, cnt_acc):
    s = pl.program_id(1)
    ns = pl.num_programs(1)
    d = A_ref.shape[1]

    @pl.when(s == 0)
    def _zero():
        S_acc[...] = jnp.zeros_like(S_acc)
        cnt_acc[...] = jnp.zeros_like(cnt_acc)

    sx = spad_ref[:, 0:1]
    sy = spad_ref[:, 1:2]
    dx = dpT_ref[0:1, :]
    dy = dpT_ref[1:2, :]

    overlap = jnp.logical_and(jnp.min(sx) <= jnp.max(dx) + _THR,
                              jnp.max(sx) >= jnp.min(dx) - _THR)

    @pl.when(overlap)
    def _tile():
        diffx = sx - dx
        diffy = sy - dy
        dist = jnp.sqrt(diffx * diffx + diffy * diffy)
        mask = (dist <= _THR).astype(jnp.float32)

        u = u_ref[...]
        v = v_ref[...]
        ed = jnp.maximum(u[:, None, :] - v[None, :, :], 0.0)
        M = jax.lax.dot_general(
            ed.reshape(_SRC_SUB * _DST_BLK, d), W_e1b_ref[...],
            (((1,), (0,)), ((), ())), preferred_element_type=jnp.float32)
        hidden = jnp.maximum(
            M.reshape(_SRC_SUB, _DST_BLK, d)
            + A_ref[...][:, None, :] + C_ref[...][None, :, :], 0.0)
        S_acc[...] += jnp.sum(mask[:, :, None] * hidden, axis=0)
        ones = jnp.ones((_SRC_SUB, 128), jnp.float32)
        cnt_acc[...] += jax.lax.dot_general(
            mask, ones, (((0,), (0,)), ((), ())),
            preferred_element_type=jnp.float32)

    @pl.when(s == ns - 1)
    def _finalize():
        cnt_col = cnt_acc[:, 0:1]
        acc = (dst_enc_ref[...] + S_acc[...] @ W_e2_ref[...]
               + cnt_col * b_e2_ref[...])
        mean = jnp.mean(acc, axis=1, keepdims=True)
        cen = acc - mean
        var = jnp.mean(cen * cen, axis=1, keepdims=True)
        nrm = cen / jnp.sqrt(var + 1e-5) * ln_g_ref[...] + ln_b_ref[...]
        h = jnp.maximum(nrm, 0.0)
        o = h @ W_out_ref[...] + b_out_ref[...] + dst_feat_ref[...]
        out_ref[...] = jnp.maximum(o, 0.0)


@jax.jit
def kernel(src_node_features, src_node_pos, dst_node_features, dst_node_pos,
           W_src, b_src, W_dst, b_dst, W_ed, b_ed,
           W_e1, b_e1, W_e2, b_e2, ln_gamma, ln_beta, W_out, b_out):
    src_n, d = src_node_features.shape
    dst_n = dst_node_features.shape[0]
    f32 = jnp.float32

    W_e1a = W_e1[0:d]
    W_e1b = W_e1[d:2 * d]
    W_e1c = W_e1[2 * d:]
    b_src_r = b_src.reshape(1, d)
    b_dst_r = b_dst.reshape(1, d)
    b_e1_r = b_e1.reshape(1, d)
    b_ed_r = b_ed.reshape(1, d)
    b_e2_r = b_e2.reshape(1, d)
    ln_g_r = ln_gamma.reshape(1, d)
    ln_b_r = ln_beta.reshape(1, d)
    b_out_r = b_out.reshape(1, d)

    sperm = jnp.argsort(src_node_pos[:, 0])
    dperm = jnp.argsort(dst_node_pos[:, 0])
    sf_s = jnp.take(src_node_features, sperm, axis=0)
    sp_s = jnp.take(src_node_pos, sperm, axis=0)
    df_s = jnp.take(dst_node_features, dperm, axis=0)
    dp_s = jnp.take(dst_node_pos, dperm, axis=0)

    spad = jnp.pad(sp_s, ((0, 0), (0, _POS_PAD - 2)))
    dpT = jnp.pad(dp_s.T, ((0, 6), (0, 0)))

    A, u, C, v, dst_enc = pl.pallas_call(
        _prep_kernel,
        out_shape=[
            jax.ShapeDtypeStruct((src_n, d), f32),
            jax.ShapeDtypeStruct((src_n, d), f32),
            jax.ShapeDtypeStruct((dst_n, d), f32),
            jax.ShapeDtypeStruct((dst_n, d), f32),
            jax.ShapeDtypeStruct((dst_n, d), f32),
        ],
    )(sf_s, sp_s, df_s, dp_s,
      W_src, b_src_r, W_dst, b_dst_r, W_ed, b_ed_r, W_e1a, b_e1_r, W_e1c)

    db = dst_n // _DST_BLK
    sb = src_n // _SRC_SUB
    out = pl.pallas_call(
        _main_kernel,
        grid=(db, sb),
        in_specs=[
            pl.BlockSpec((_SRC_SUB, d), lambda i, j: (j, 0)),
            pl.BlockSpec((_SRC_SUB, d), lambda i, j: (j, 0)),
            pl.BlockSpec((_SRC_SUB, _POS_PAD), lambda i, j: (j, 0)),
            pl.BlockSpec((_DST_BLK, d), lambda i, j: (i, 0)),
            pl.BlockSpec((_DST_BLK, d), lambda i, j: (i, 0)),
            pl.BlockSpec((8, _DST_BLK), lambda i, j: (0, i)),
            pl.BlockSpec((_DST_BLK, d), lambda i, j: (i, 0)),
            pl.BlockSpec((_DST_BLK, d), lambda i, j: (i, 0)),
            pl.BlockSpec((d, d), lambda i, j: (0, 0)),
            pl.BlockSpec((d, d), lambda i, j: (0, 0)),
            pl.BlockSpec((1, d), lambda i, j: (0, 0)),
            pl.BlockSpec((1, d), lambda i, j: (0, 0)),
            pl.BlockSpec((1, d), lambda i, j: (0, 0)),
            pl.BlockSpec((d, d), lambda i, j: (0, 0)),
            pl.BlockSpec((1, d), lambda i, j: (0, 0)),
        ],
        out_specs=pl.BlockSpec((_DST_BLK, d), lambda i, j: (i, 0)),
        out_shape=jax.ShapeDtypeStruct((dst_n, d), f32),
        scratch_shapes=[
            pltpu.VMEM((_DST_BLK, d), f32),
            pltpu.VMEM((_DST_BLK, 128), f32),
        ],
        compiler_params=pltpu.CompilerParams(
            dimension_semantics=("parallel", "arbitrary")),
    )(A, u, spad, C, v, dpT, dst_enc, df_s,
      W_e1b, W_e2, b_e2_r, ln_g_r, ln_b_r, W_out, b_out_r)
    return jnp.take(out, jnp.argsort(dperm), axis=0)


# speedup vs baseline: 29.2132x; 1.2991x over previous
import jax
import jax.numpy as jnp
from jax.experimental import pallas as pl
from jax.experimental.pallas import tpu as pltpu

_THR = 5.0
_SRC_SUB = 64
_DST_BLK = 256
_POS_PAD = 128


def _prep_kernel(sf_ref, sp_ref, df_ref, dp_ref,
                 W_src_ref, b_src_ref, W_dst_ref, b_dst_ref,
                 W_ed_ref, b_ed_ref, W_e1a_ref, b_e1_ref, W_e1c_ref,
                 A_ref, u_ref, C_ref, v_ref, dst_enc_ref):
    src_enc = jnp.maximum(sf_ref[...] @ W_src_ref[...] + b_src_ref[...], 0.0)
    A_ref[...] = src_enc @ W_e1a_ref[...] + b_e1_ref[...]
    sp = sp_ref[...]
    W_ed = W_ed_ref[...]
    u_ref[...] = (sp[:, 0:1] * W_ed[0:1, :] + sp[:, 1:2] * W_ed[1:2, :]
                  + b_ed_ref[...])
    dst_enc = jnp.maximum(df_ref[...] @ W_dst_ref[...] + b_dst_ref[...], 0.0)
    dst_enc_ref[...] = dst_enc
    C_ref[...] = dst_enc @ W_e1c_ref[...]
    dp = dp_ref[...]
    v_ref[...] = dp[:, 0:1] * W_ed[0:1, :] + dp[:, 1:2] * W_ed[1:2, :]


def _main_kernel(A_ref, u_ref, spad_ref, C_ref, v_ref, dpT_ref,
                 dst_enc_ref, dst_feat_ref,
                 W_e1b_ref, W_e2_ref, b_e2_ref,
                 ln_g_ref, ln_b_ref, W_out_ref, b_out_ref,
                 out_ref, S_acc, cnt_acc):
    s = pl.program_id(1)
    ns = pl.num_programs(1)
    d = A_ref.shape[1]

    @pl.when(s == 0)
    def _zero():
        S_acc[...] = jnp.zeros_like(S_acc)
        cnt_acc[...] = jnp.zeros_like(cnt_acc)

    sx = spad_ref[:, 0:1]
    sy = spad_ref[:, 1:2]
    dx = dpT_ref[0:1, :]
    dy = dpT_ref[1:2, :]

    overlap = jnp.logical_and(jnp.min(sx) <= jnp.max(dx) + _THR,
                              jnp.max(sx) >= jnp.min(dx) - _THR)

    @pl.when(overlap)
    def _tile():
        diffx = sx - dx
        diffy = sy - dy
        dist = jnp.sqrt(diffx * diffx + diffy * diffy)
        mask = (dist <= _THR).astype(jnp.float32)

        u = u_ref[...]
        v = v_ref[...]
        ed = jnp.maximum(u[:, None, :] - v[None, :, :], 0.0)
        M = jax.lax.dot_general(
            ed.reshape(_SRC_SUB * _DST_BLK, d), W_e1b_ref[...],
            (((1,), (0,)), ((), ())), preferred_element_type=jnp.float32)
        hidden = jnp.maximum(
            M.reshape(_SRC_SUB, _DST_BLK, d)
            + A_ref[...][:, None, :] + C_ref[...][None, :, :], 0.0)
        S_acc[...] += jnp.sum(mask[:, :, None] * hidden, axis=0)
        ones = jnp.ones((_SRC_SUB, 128), jnp.float32)
        cnt_acc[...] += jax.lax.dot_general(
            mask, ones, (((0,), (0,)), ((), ())),
            preferred_element_type=jnp.float32)

    @pl.when(s == ns - 1)
    def _finalize():
        cnt_col = cnt_acc[:, 0:1]
        acc = (dst_enc_ref[...] + S_acc[...] @ W_e2_ref[...]
               + cnt_col * b_e2_ref[...])
        mean = jnp.mean(acc, axis=1, keepdims=True)
        cen = acc - mean
        var = jnp.mean(cen * cen, axis=1, keepdims=True)
        nrm = cen / jnp.sqrt(var + 1e-5) * ln_g_ref[...] + ln_b_ref[...]
        h = jnp.maximum(nrm, 0.0)
        o = h @ W_out_ref[...] + b_out_ref[...] + dst_feat_ref[...]
        out_ref[...] = jnp.maximum(o, 0.0)


@jax.jit
def kernel(src_node_features, src_node_pos, dst_node_features, dst_node_pos,
           W_src, b_src, W_dst, b_dst, W_ed, b_ed,
           W_e1, b_e1, W_e2, b_e2, ln_gamma, ln_beta, W_out, b_out):
    src_n, d = src_node_features.shape
    dst_n = dst_node_features.shape[0]
    f32 = jnp.float32

    W_e1a = W_e1[0:d]
    W_e1b = W_e1[d:2 * d]
    W_e1c = W_e1[2 * d:]
    b_src_r = b_src.reshape(1, d)
    b_dst_r = b_dst.reshape(1, d)
    b_e1_r = b_e1.reshape(1, d)
    b_ed_r = b_ed.reshape(1, d)
    b_e2_r = b_e2.reshape(1, d)
    ln_g_r = ln_gamma.reshape(1, d)
    ln_b_r = ln_beta.reshape(1, d)
    b_out_r = b_out.reshape(1, d)

    sperm = jnp.argsort(src_node_pos[:, 0])
    dperm = jnp.argsort(dst_node_pos[:, 0])
    sf_s = jnp.take(src_node_features, sperm, axis=0)
    sp_s = jnp.take(src_node_pos, sperm, axis=0)
    df_s = jnp.take(dst_node_features, dperm, axis=0)
    dp_s = jnp.take(dst_node_pos, dperm, axis=0)

    spad = jnp.pad(sp_s, ((0, 0), (0, _POS_PAD - 2)))
    dpT = jnp.pad(dp_s.T, ((0, 6), (0, 0)))

    A, u, C, v, dst_enc = pl.pallas_call(
        _prep_kernel,
        out_shape=[
            jax.ShapeDtypeStruct((src_n, d), f32),
            jax.ShapeDtypeStruct((src_n, d), f32),
            jax.ShapeDtypeStruct((dst_n, d), f32),
            jax.ShapeDtypeStruct((dst_n, d), f32),
            jax.ShapeDtypeStruct((dst_n, d), f32),
        ],
    )(sf_s, sp_s, df_s, dp_s,
      W_src, b_src_r, W_dst, b_dst_r, W_ed, b_ed_r, W_e1a, b_e1_r, W_e1c)

    db = dst_n // _DST_BLK
    sb = src_n // _SRC_SUB
    out = pl.pallas_call(
        _main_kernel,
        grid=(db, sb),
        in_specs=[
            pl.BlockSpec((_SRC_SUB, d), lambda i, j: (j, 0)),
            pl.BlockSpec((_SRC_SUB, d), lambda i, j: (j, 0)),
            pl.BlockSpec((_SRC_SUB, _POS_PAD), lambda i, j: (j, 0)),
            pl.BlockSpec((_DST_BLK, d), lambda i, j: (i, 0)),
            pl.BlockSpec((_DST_BLK, d), lambda i, j: (i, 0)),
            pl.BlockSpec((8, _DST_BLK), lambda i, j: (0, i)),
            pl.BlockSpec((_DST_BLK, d), lambda i, j: (i, 0)),
            pl.BlockSpec((_DST_BLK, d), lambda i, j: (i, 0)),
            pl.BlockSpec((d, d), lambda i, j: (0, 0)),
            pl.BlockSpec((d, d), lambda i, j: (0, 0)),
            pl.BlockSpec((1, d), lambda i, j: (0, 0)),
            pl.BlockSpec((1, d), lambda i, j: (0, 0)),
            pl.BlockSpec((1, d), lambda i, j: (0, 0)),
            pl.BlockSpec((d, d), lambda i, j: (0, 0)),
            pl.BlockSpec((1, d), lambda i, j: (0, 0)),
        ],
        out_specs=pl.BlockSpec((_DST_BLK, d), lambda i, j: (i, 0)),
        out_shape=jax.ShapeDtypeStruct((dst_n, d), f32),
        scratch_shapes=[
            pltpu.VMEM((_DST_BLK, d), f32),
            pltpu.VMEM((_DST_BLK, 128), f32),
        ],
        compiler_params=pltpu.CompilerParams(
            dimension_semantics=("parallel", "arbitrary")),
    )(A, u, spad, C, v, dpT, dst_enc, df_s,
      W_e1b, W_e2, b_e2_r, ln_g_r, ln_b_r, W_out, b_out_r)
    return jnp.take(out, jnp.argsort(dperm), axis=0)


# speedup vs baseline: 33.1346x; 1.1342x over previous
import jax
import jax.numpy as jnp
from jax.experimental import pallas as pl
from jax.experimental.pallas import tpu as pltpu

_THR = 5.0
_SRC_SUB = 128
_DST_BLK = 256
_POS_PAD = 128


def _prep_kernel(sf_ref, sp_ref, df_ref, dp_ref,
                 W_src_ref, b_src_ref, W_dst_ref, b_dst_ref,
                 W_ed_ref, b_ed_ref, W_e1a_ref, b_e1_ref, W_e1c_ref,
                 A_ref, u_ref, C_ref, v_ref, dst_enc_ref):
    src_enc = jnp.maximum(sf_ref[...] @ W_src_ref[...] + b_src_ref[...], 0.0)
    A_ref[...] = src_enc @ W_e1a_ref[...] + b_e1_ref[...]
    sp = sp_ref[...]
    W_ed = W_ed_ref[...]
    u_ref[...] = (sp[:, 0:1] * W_ed[0:1, :] + sp[:, 1:2] * W_ed[1:2, :]
                  + b_ed_ref[...])
    dst_enc = jnp.maximum(df_ref[...] @ W_dst_ref[...] + b_dst_ref[...], 0.0)
    dst_enc_ref[...] = dst_enc
    C_ref[...] = dst_enc @ W_e1c_ref[...]
    dp = dp_ref[...]
    v_ref[...] = dp[:, 0:1] * W_ed[0:1, :] + dp[:, 1:2] * W_ed[1:2, :]


def _main_kernel(A_ref, u_ref, spad_ref, C_ref, v_ref, dpT_ref,
                 dst_enc_ref, dst_feat_ref,
                 W_e1b_ref, W_e2_ref, b_e2_ref,
                 ln_g_ref, ln_b_ref, W_out_ref, b_out_ref,
                 out_ref, S_acc, cnt_acc):
    s = pl.program_id(1)
    ns = pl.num_programs(1)
    d = A_ref.shape[1]

    @pl.when(s == 0)
    def _zero():
        S_acc[...] = jnp.zeros_like(S_acc)
        cnt_acc[...] = jnp.zeros_like(cnt_acc)

    sx = spad_ref[:, 0:1]
    sy = spad_ref[:, 1:2]
    dx = dpT_ref[0:1, :]
    dy = dpT_ref[1:2, :]

    overlap = jnp.logical_and(jnp.min(sx) <= jnp.max(dx) + _THR,
                              jnp.max(sx) >= jnp.min(dx) - _THR)

    @pl.when(overlap)
    def _tile():
        diffx = sx - dx
        diffy = sy - dy
        dist = jnp.sqrt(diffx * diffx + diffy * diffy)
        mask = (dist <= _THR).astype(jnp.float32)

        u = u_ref[...]
        v = v_ref[...]
        ed = jnp.maximum(u[:, None, :] - v[None, :, :], 0.0)
        M = jax.lax.dot_general(
            ed.reshape(_SRC_SUB * _DST_BLK, d), W_e1b_ref[...],
            (((1,), (0,)), ((), ())), preferred_element_type=jnp.float32)
        hidden = jnp.maximum(
            M.reshape(_SRC_SUB, _DST_BLK, d)
            + A_ref[...][:, None, :] + C_ref[...][None, :, :], 0.0)
        S_acc[...] += jnp.sum(mask[:, :, None] * hidden, axis=0)
        ones = jnp.ones((_SRC_SUB, 128), jnp.float32)
        cnt_acc[...] += jax.lax.dot_general(
            mask, ones, (((0,), (0,)), ((), ())),
            preferred_element_type=jnp.float32)

    @pl.when(s == ns - 1)
    def _finalize():
        cnt_col = cnt_acc[:, 0:1]
        acc = (dst_enc_ref[...] + S_acc[...] @ W_e2_ref[...]
               + cnt_col * b_e2_ref[...])
        mean = jnp.mean(acc, axis=1, keepdims=True)
        cen = acc - mean
        var = jnp.mean(cen * cen, axis=1, keepdims=True)
        nrm = cen / jnp.sqrt(var + 1e-5) * ln_g_ref[...] + ln_b_ref[...]
        h = jnp.maximum(nrm, 0.0)
        o = h @ W_out_ref[...] + b_out_ref[...] + dst_feat_ref[...]
        out_ref[...] = jnp.maximum(o, 0.0)


@jax.jit
def kernel(src_node_features, src_node_pos, dst_node_features, dst_node_pos,
           W_src, b_src, W_dst, b_dst, W_ed, b_ed,
           W_e1, b_e1, W_e2, b_e2, ln_gamma, ln_beta, W_out, b_out):
    src_n, d = src_node_features.shape
    dst_n = dst_node_features.shape[0]
    f32 = jnp.float32

    W_e1a = W_e1[0:d]
    W_e1b = W_e1[d:2 * d]
    W_e1c = W_e1[2 * d:]
    b_src_r = b_src.reshape(1, d)
    b_dst_r = b_dst.reshape(1, d)
    b_e1_r = b_e1.reshape(1, d)
    b_ed_r = b_ed.reshape(1, d)
    b_e2_r = b_e2.reshape(1, d)
    ln_g_r = ln_gamma.reshape(1, d)
    ln_b_r = ln_beta.reshape(1, d)
    b_out_r = b_out.reshape(1, d)

    sperm = jnp.argsort(src_node_pos[:, 0])
    dperm = jnp.argsort(dst_node_pos[:, 0])
    sf_s = jnp.take(src_node_features, sperm, axis=0)
    sp_s = jnp.take(src_node_pos, sperm, axis=0)
    df_s = jnp.take(dst_node_features, dperm, axis=0)
    dp_s = jnp.take(dst_node_pos, dperm, axis=0)

    spad = jnp.pad(sp_s, ((0, 0), (0, _POS_PAD - 2)))
    dpT = jnp.pad(dp_s.T, ((0, 6), (0, 0)))

    A, u, C, v, dst_enc = pl.pallas_call(
        _prep_kernel,
        out_shape=[
            jax.ShapeDtypeStruct((src_n, d), f32),
            jax.ShapeDtypeStruct((src_n, d), f32),
            jax.ShapeDtypeStruct((dst_n, d), f32),
            jax.ShapeDtypeStruct((dst_n, d), f32),
            jax.ShapeDtypeStruct((dst_n, d), f32),
        ],
    )(sf_s, sp_s, df_s, dp_s,
      W_src, b_src_r, W_dst, b_dst_r, W_ed, b_ed_r, W_e1a, b_e1_r, W_e1c)

    db = dst_n // _DST_BLK
    sb = src_n // _SRC_SUB
    out = pl.pallas_call(
        _main_kernel,
        grid=(db, sb),
        in_specs=[
            pl.BlockSpec((_SRC_SUB, d), lambda i, j: (j, 0)),
            pl.BlockSpec((_SRC_SUB, d), lambda i, j: (j, 0)),
            pl.BlockSpec((_SRC_SUB, _POS_PAD), lambda i, j: (j, 0)),
            pl.BlockSpec((_DST_BLK, d), lambda i, j: (i, 0)),
            pl.BlockSpec((_DST_BLK, d), lambda i, j: (i, 0)),
            pl.BlockSpec((8, _DST_BLK), lambda i, j: (0, i)),
            pl.BlockSpec((_DST_BLK, d), lambda i, j: (i, 0)),
            pl.BlockSpec((_DST_BLK, d), lambda i, j: (i, 0)),
            pl.BlockSpec((d, d), lambda i, j: (0, 0)),
            pl.BlockSpec((d, d), lambda i, j: (0, 0)),
            pl.BlockSpec((1, d), lambda i, j: (0, 0)),
            pl.BlockSpec((1, d), lambda i, j: (0, 0)),
            pl.BlockSpec((1, d), lambda i, j: (0, 0)),
            pl.BlockSpec((d, d), lambda i, j: (0, 0)),
            pl.BlockSpec((1, d), lambda i, j: (0, 0)),
        ],
        out_specs=pl.BlockSpec((_DST_BLK, d), lambda i, j: (i, 0)),
        out_shape=jax.ShapeDtypeStruct((dst_n, d), f32),
        scratch_shapes=[
            pltpu.VMEM((_DST_BLK, d), f32),
            pltpu.VMEM((_DST_BLK, 128), f32),
        ],
        compiler_params=pltpu.CompilerParams(
            dimension_semantics=("parallel", "arbitrary")),
    )(A, u, spad, C, v, dpT, dst_enc, df_s,
      W_e1b, W_e2, b_e2_r, ln_g_r, ln_b_r, W_out, b_out_r)
    return jnp.take(out, jnp.argsort(dperm), axis=0)
